# Initial kernel scaffold; baseline (speedup 1.0000x reference)
import jax
import jax.numpy as jnp
from jax.experimental import pallas as pl

HEADS = 16
GS = 4
NG = HEADS // GS
QL = 2048
KL = 2048
HEAVY = 204
RECENT = 204
BQ = 256

F32_MIN = float(jnp.finfo(jnp.float32).min)
_SUM_RECENT = float(RECENT * (RECENT + 1) // 2 + (QL - RECENT) * (RECENT + 1))
_WMAX = float(QL - RECENT - 1)


def _importance_kernel(x_ref, imp_ref):
    qb = pl.program_id(1)
    x = x_ref[0, :, :]
    m = jnp.max(x, axis=-1, keepdims=True)
    e = jnp.exp(x - m)
    s = jnp.sum(e, axis=-1, keepdims=True)
    contrib = jnp.sum(e / s, axis=0, keepdims=True)

    @pl.when(qb == 0)
    def _():
        imp_ref[...] = jnp.zeros((1, 1, KL), jnp.float32)

    imp_ref[...] += contrib.reshape(1, 1, KL)


def _select_kernel(imp_ref, heavy_ref, dens_ref):
    v = imp_ref[:, 0, :]
    u = jax.lax.bitcast_convert_type(v, jnp.int32)

    def bit_step(i, p):
        cand = p | (jnp.int32(1) << (jnp.int32(30) - i))
        cnt = jnp.sum((u >= cand).astype(jnp.int32), axis=-1, keepdims=True)
        return jnp.where(cnt >= HEAVY, cand, p)

    p = jax.lax.fori_loop(0, 31, bit_step, jnp.zeros((HEADS, 1), jnp.int32))

    cnt_gt = jnp.sum((u > p).astype(jnp.int32), axis=-1, keepdims=True)
    need = HEAVY - cnt_gt
    eq = u == p
    ik = jax.lax.broadcasted_iota(jnp.int32, (HEADS, KL), 1)

    def idx_step(i, s):
        cand = s | (jnp.int32(1) << (jnp.int32(10) - i))
        c = jnp.sum((eq & (ik < cand)).astype(jnp.int32), axis=-1,
                    keepdims=True)
        return jnp.where(c < need, cand, s)

    s = jax.lax.fori_loop(0, 11, idx_step, jnp.zeros((HEADS, 1), jnp.int32))

    heavy16 = ((u > p) | (eq & (ik <= s))).astype(jnp.float32)

    gh = jax.lax.broadcasted_iota(jnp.int32, (NG, HEADS), 1)
    gg = jax.lax.broadcasted_iota(jnp.int32, (NG, HEADS), 0)
    gmat = (gh // GS == gg).astype(jnp.float32)
    sg = jnp.dot(gmat, heavy16, preferred_element_type=jnp.float32)
    hg = (sg > 0.0).astype(jnp.float32)
    heavy_ref[...] = hg.reshape(NG, 1, KL)

    ikg = jax.lax.broadcasted_iota(jnp.int32, (NG, KL), 1).astype(jnp.float32)
    w = jnp.maximum(0.0, _WMAX - ikg)
    count_g = _SUM_RECENT + jnp.sum(hg * w, axis=-1)
    total = GS * jnp.sum(count_g)
    dens_ref[0, 0] = total / HEADS / (QL * (QL + 1) / 2.0)


def _mask_kernel(hv_ref, out_ref):
    qb = pl.program_id(1)
    hv = hv_ref[0, :, :] > 0.0
    iq = qb * BQ + jax.lax.broadcasted_iota(jnp.int32, (BQ, 1), 0)
    ik = jax.lax.broadcasted_iota(jnp.int32, (1, KL), 1)
    keep = (ik <= iq) & (hv | (ik >= iq - RECENT))
    out_ref[...] = jnp.where(keep, 0.0, F32_MIN).reshape(1, BQ, KL)


def kernel(attn_weights, group_size):
    x = attn_weights.reshape(HEADS, QL, KL)

    imp = pl.pallas_call(
        _importance_kernel,
        grid=(HEADS, QL // BQ),
        in_specs=[pl.BlockSpec((1, BQ, KL), lambda h, q: (h, q, 0))],
        out_specs=pl.BlockSpec((1, 1, KL), lambda h, q: (h, 0, 0)),
        out_shape=jax.ShapeDtypeStruct((HEADS, 1, KL), jnp.float32),
    )(x)

    hg, dens = pl.pallas_call(
        _select_kernel,
        out_shape=[
            jax.ShapeDtypeStruct((NG, 1, KL), jnp.float32),
            jax.ShapeDtypeStruct((1, 1), jnp.float32),
        ],
    )(imp)

    mask = pl.pallas_call(
        _mask_kernel,
        grid=(HEADS, QL // BQ),
        in_specs=[pl.BlockSpec((1, 1, KL), lambda h, q: (h // GS, 0, 0))],
        out_specs=pl.BlockSpec((1, BQ, KL), lambda h, q: (h, q, 0)),
        out_shape=jax.ShapeDtypeStruct((HEADS, QL, KL), jnp.float32),
    )(hg)

    density = dens.reshape(())
    density = density + (jnp.asarray(group_size) - GS).astype(jnp.float32) * 0.0
    return (mask.reshape(1, HEADS, QL, KL), density)


# speedup vs baseline: 1.5438x; 1.5438x over previous
import jax
import jax.numpy as jnp
from jax.experimental import pallas as pl

HEADS = 16
GS = 4
NG = HEADS // GS
QL = 2048
KL = 2048
HEAVY = 204
RECENT = 204
BQ = 256

F32_MIN = float(jnp.finfo(jnp.float32).min)
_SUM_RECENT = float(RECENT * (RECENT + 1) // 2 + (QL - RECENT) * (RECENT + 1))
_WMAX = float(QL - RECENT - 1)


def _importance_kernel(x_ref, imp_ref):
    qb = pl.program_id(1)
    x = x_ref[0, :, :]
    m = jnp.max(x, axis=-1, keepdims=True)
    e = jnp.exp(x - m)
    s = jnp.sum(e, axis=-1, keepdims=True)
    contrib = jnp.sum(e / s, axis=0, keepdims=True)

    @pl.when(qb == 0)
    def _():
        imp_ref[...] = jnp.zeros((1, 1, KL), jnp.float32)

    imp_ref[...] += contrib.reshape(1, 1, KL)


def _select_kernel(imp_ref, heavy_ref, dens_ref):
    v = imp_ref[:, 0, :]
    u = jax.lax.bitcast_convert_type(v, jnp.int32)

    def bit_step(i, p):
        cand = p | (jnp.int32(1) << (jnp.int32(30) - i))
        cnt = jnp.sum((u >= cand).astype(jnp.int32), axis=-1, keepdims=True)
        return jnp.where(cnt >= HEAVY, cand, p)

    p = jax.lax.fori_loop(0, 31, bit_step, jnp.zeros((HEADS, 1), jnp.int32))

    cnt_gt = jnp.sum((u > p).astype(jnp.int32), axis=-1, keepdims=True)
    need = HEAVY - cnt_gt
    eq = u == p
    ik = jax.lax.broadcasted_iota(jnp.int32, (HEADS, KL), 1)

    def idx_step(i, s):
        cand = s | (jnp.int32(1) << (jnp.int32(10) - i))
        c = jnp.sum((eq & (ik < cand)).astype(jnp.int32), axis=-1,
                    keepdims=True)
        return jnp.where(c < need, cand, s)

    s = jax.lax.fori_loop(0, 11, idx_step, jnp.zeros((HEADS, 1), jnp.int32))

    heavy16 = ((u > p) | (eq & (ik <= s))).astype(jnp.float32)

    gh = jax.lax.broadcasted_iota(jnp.int32, (NG, HEADS), 1)
    gg = jax.lax.broadcasted_iota(jnp.int32, (NG, HEADS), 0)
    gmat = (gh // GS == gg).astype(jnp.float32)
    sg = jnp.dot(gmat, heavy16, preferred_element_type=jnp.float32)
    hg = (sg > 0.0).astype(jnp.float32)
    heavy_ref[...] = hg.reshape(NG, 1, KL)

    ikg = jax.lax.broadcasted_iota(jnp.int32, (NG, KL), 1).astype(jnp.float32)
    w = jnp.maximum(0.0, _WMAX - ikg)
    count_g = _SUM_RECENT + jnp.sum(hg * w, axis=-1, keepdims=True)
    total = GS * jnp.sum(count_g, axis=0, keepdims=True)
    dens_ref[...] = total / HEADS / (QL * (QL + 1) / 2.0)


def _mask_kernel(hv_ref, out_ref):
    qb = pl.program_id(1)
    hv = hv_ref[0, :, :] > 0.0
    iq = qb * BQ + jax.lax.broadcasted_iota(jnp.int32, (BQ, 1), 0)
    ik = jax.lax.broadcasted_iota(jnp.int32, (1, KL), 1)
    keep = (ik <= iq) & (hv | (ik >= iq - RECENT))
    out_ref[...] = jnp.where(keep, 0.0, F32_MIN).reshape(1, BQ, KL)


def kernel(attn_weights, group_size):
    x = attn_weights.reshape(HEADS, QL, KL)

    imp = pl.pallas_call(
        _importance_kernel,
        grid=(HEADS, QL // BQ),
        in_specs=[pl.BlockSpec((1, BQ, KL), lambda h, q: (h, q, 0))],
        out_specs=pl.BlockSpec((1, 1, KL), lambda h, q: (h, 0, 0)),
        out_shape=jax.ShapeDtypeStruct((HEADS, 1, KL), jnp.float32),
    )(x)

    hg, dens = pl.pallas_call(
        _select_kernel,
        out_shape=[
            jax.ShapeDtypeStruct((NG, 1, KL), jnp.float32),
            jax.ShapeDtypeStruct((1, 1), jnp.float32),
        ],
    )(imp)

    mask = pl.pallas_call(
        _mask_kernel,
        grid=(HEADS, QL // BQ),
        in_specs=[pl.BlockSpec((1, 1, KL), lambda h, q: (h // GS, 0, 0))],
        out_specs=pl.BlockSpec((1, BQ, KL), lambda h, q: (h, q, 0)),
        out_shape=jax.ShapeDtypeStruct((HEADS, QL, KL), jnp.float32),
    )(hg)

    density = dens.reshape(())
    density = density + (jnp.asarray(group_size) - GS).astype(jnp.float32) * 0.0
    return (mask.reshape(1, HEADS, QL, KL), density)


# speedup vs baseline: 2.1751x; 1.4089x over previous
import jax
import jax.numpy as jnp
from jax.experimental import pallas as pl
from jax.experimental.pallas import tpu as pltpu

HEADS = 16
GS = 4
NG = HEADS // GS
QL = 2048
KL = 2048
HEAVY = 204
RECENT = 204
BQ = 256
QB = QL // BQ

F32_MIN = float(jnp.finfo(jnp.float32).min)
_SUM_RECENT = float(RECENT * (RECENT + 1) // 2 + (QL - RECENT) * (RECENT + 1))
_WMAX = float(QL - RECENT - 1)
_DSCALE = GS / HEADS / (QL * (QL + 1) / 2.0)


def _fused_kernel(x_ref, mask_ref, dens_ref, acc_ref, heavy_ref):
    i = pl.program_id(0)
    qb = pl.program_id(1)

    @pl.when((i > 0) & (qb == 0))
    def _select():
        v = acc_ref[:, 0, :]
        u = jax.lax.bitcast_convert_type(v, jnp.int32)

        def bit_step(j, p):
            cand = p | (jnp.int32(1) << (jnp.int32(30) - j))
            cnt = jnp.sum((u >= cand).astype(jnp.int32), axis=-1,
                          keepdims=True)
            return jnp.where(cnt >= HEAVY, cand, p)

        p = jax.lax.fori_loop(0, 31, bit_step, jnp.zeros((GS, 1), jnp.int32))

        cnt_gt = jnp.sum((u > p).astype(jnp.int32), axis=-1, keepdims=True)
        need = HEAVY - cnt_gt
        eq = u == p
        ik = jax.lax.broadcasted_iota(jnp.int32, (GS, KL), 1)

        def idx_step(j, t):
            cand = t | (jnp.int32(1) << (jnp.int32(10) - j))
            c = jnp.sum((eq & (ik < cand)).astype(jnp.int32), axis=-1,
                        keepdims=True)
            return jnp.where(c < need, cand, t)

        t = jax.lax.fori_loop(0, 11, idx_step, jnp.zeros((GS, 1), jnp.int32))

        heavy4 = ((u > p) | (eq & (ik <= t))).astype(jnp.float32)
        hg = jnp.max(heavy4, axis=0, keepdims=True)
        heavy_ref[...] = hg

        ikf = jax.lax.broadcasted_iota(jnp.int32, (1, KL), 1).astype(
            jnp.float32)
        w = jnp.maximum(0.0, _WMAX - ikf)
        count_g = _SUM_RECENT + jnp.sum(hg * w, axis=-1, keepdims=True)
        contrib = count_g * _DSCALE
        dens_ref[...] = jnp.where(i == 1, contrib, dens_ref[...] + contrib)

    @pl.when(i < NG)
    def _importance():
        x = x_ref[...]
        m = jnp.max(x, axis=-1, keepdims=True)
        e = jnp.exp(x - m)
        s = jnp.sum(e, axis=-1, keepdims=True)
        contrib = jnp.sum(e / s, axis=1)[:, None, :]

        @pl.when(qb == 0)
        def _():
            acc_ref[...] = contrib

        @pl.when(qb != 0)
        def _():
            acc_ref[...] += contrib

    @pl.when(i > 0)
    def _mask():
        hv = heavy_ref[...] > 0.0
        iq = qb * BQ + jax.lax.broadcasted_iota(jnp.int32, (BQ, 1), 0)
        ik = jax.lax.broadcasted_iota(jnp.int32, (1, KL), 1)
        keep = (ik <= iq) & (hv | (ik >= iq - RECENT))
        blk = jnp.where(keep, 0.0, F32_MIN)
        mask_ref[...] = jnp.broadcast_to(blk[None], (GS, BQ, KL))


def kernel(attn_weights, group_size):
    x = attn_weights.reshape(HEADS, QL, KL)

    mask, dens = pl.pallas_call(
        _fused_kernel,
        grid=(NG + 1, QB),
        in_specs=[
            pl.BlockSpec(
                (GS, BQ, KL),
                lambda i, qb: (jnp.where(i < NG, i, 0),
                               jnp.where(i < NG, qb, 0), 0),
            ),
        ],
        out_specs=[
            pl.BlockSpec(
                (GS, BQ, KL),
                lambda i, qb: (jnp.where(i > 0, i - 1, 0),
                               jnp.where(i > 0, qb, 0), 0),
            ),
            pl.BlockSpec((1, 1), lambda i, qb: (0, 0)),
        ],
        out_shape=[
            jax.ShapeDtypeStruct((HEADS, QL, KL), jnp.float32),
            jax.ShapeDtypeStruct((1, 1), jnp.float32),
        ],
        scratch_shapes=[
            pltpu.VMEM((GS, 1, KL), jnp.float32),
            pltpu.VMEM((1, KL), jnp.float32),
        ],
    )(x)

    density = dens.reshape(())
    density = density + (jnp.asarray(group_size) - GS).astype(jnp.float32) * 0.0
    return (mask.reshape(1, HEADS, QL, KL), density)
